# Initial kernel scaffold; baseline (speedup 1.0000x reference)
import jax
import jax.numpy as jnp
from jax.experimental import pallas as pl


def kernel(users, items, edge_list, emb_user, emb_item):
    raise NotImplementedError("write your pallas kernel here")


# speedup vs baseline: 13.4025x; 13.4025x over previous
import functools

import jax
import jax.numpy as jnp
from jax import lax
from jax.experimental import pallas as pl
from jax.experimental.pallas import tpu as pltpu
from jax.experimental.pallas import tpu_sc as plsc

NU = 25000
NI = 25000
D = 64
NE = 800000
NLAYERS = 3
BATCH = 4096

NC = 2
NS = 16
NP = 25088
RPT = NP // NS
CH = 128
EPT = 50048
NCHUNK = EPT // CH
PAD_ROW = NP - 1
PPT = 2 * BATCH // (NC * NS)

BLK = 512


def _mesh():
    return plsc.VectorSubcoreMesh(
        core_axis_name="c", subcore_axis_name="s",
        num_cores=NC, num_subcores=NS)


_SC_PARAMS = pltpu.CompilerParams(use_tc_tiling_on_sc=False)


def _sc_degree(sidx, zdeg, ones_rows):
    def body(sidx_h, zdeg_h, ones_h, out_h, deg_sp, obuf, idxs):
        c = lax.axis_index("c")
        s = lax.axis_index("s")
        sl = pl.ds(s * RPT, RPT)
        pltpu.sync_copy(zdeg_h.at[sl], deg_sp.at[sl])
        pltpu.sync_copy(ones_h, obuf)
        plsc.subcore_barrier()
        ebase = (c * NS + s) * EPT

        def chunk(j, carry):
            pltpu.sync_copy(sidx_h.at[pl.ds(ebase + j * CH, CH)], idxs)
            pltpu.sync_copy(obuf, deg_sp.at[idxs], add=True)
            return carry

        lax.fori_loop(0, NCHUNK, chunk, 0)
        plsc.subcore_barrier()
        pltpu.sync_copy(deg_sp.at[sl], out_h.at[pl.ds(c * NP + s * RPT, RPT)])

    f = pl.kernel(
        body,
        out_type=jax.ShapeDtypeStruct((NC * NP, 16), jnp.float32),
        mesh=_mesh(),
        compiler_params=_SC_PARAMS,
        scratch_types=[
            pltpu.VMEM_SHARED((NP, 16), jnp.float32),
            pltpu.VMEM((CH, 16), jnp.float32),
            pltpu.VMEM((CH,), jnp.int32),
        ],
    )
    return f(sidx, zdeg, ones_rows)


def _sc_propagate(gtab, gidx, sidx, zrows):
    def body(gtab_h, gidx_h, sidx_h, zrows_h, out_h, acc, idxg, idxs, rows, sem):
        c = lax.axis_index("c")
        s = lax.axis_index("s")
        sl = pl.ds(s * RPT, RPT)
        pltpu.sync_copy(zrows_h.at[sl], acc.at[sl])
        plsc.subcore_barrier()
        ebase = (c * NS + s) * EPT

        def chunk(j, carry):
            off = ebase + j * CH
            pltpu.sync_copy(gidx_h.at[pl.ds(off, CH)], idxg)
            pltpu.sync_copy(sidx_h.at[pl.ds(off, CH)], idxs)
            pltpu.async_copy(gtab_h.at[idxg], rows, sem).wait()
            pltpu.sync_copy(rows, acc.at[idxs], add=True)
            return carry

        lax.fori_loop(0, NCHUNK, chunk, 0)
        plsc.subcore_barrier()
        pltpu.sync_copy(acc.at[sl], out_h.at[pl.ds(c * NP + s * RPT, RPT)])

    f = pl.kernel(
        body,
        out_type=jax.ShapeDtypeStruct((NC * NP, D), jnp.float32),
        mesh=_mesh(),
        compiler_params=_SC_PARAMS,
        scratch_types=[
            pltpu.VMEM_SHARED((NP, D), jnp.float32),
            pltpu.VMEM((CH,), jnp.int32),
            pltpu.VMEM((CH,), jnp.int32),
            pltpu.VMEM((CH, D), jnp.float32),
            pltpu.SemaphoreType.DMA,
        ],
    )
    return f(gtab, gidx, sidx, zrows)


def _sc_gather_pairs(stab, pidx):
    def body(stab_h, pidx_h, out_h, idxp, prow, sem):
        c = lax.axis_index("c")
        s = lax.axis_index("s")
        base = (c * NS + s) * PPT
        for k in range(PPT // CH):
            off = base + k * CH
            pltpu.sync_copy(pidx_h.at[pl.ds(off, CH)], idxp)
            pltpu.async_copy(stab_h.at[idxp], prow, sem).wait()
            pltpu.sync_copy(prow, out_h.at[pl.ds(off, CH)])

    f = pl.kernel(
        body,
        out_type=jax.ShapeDtypeStruct((2 * BATCH, D), jnp.float32),
        mesh=_mesh(),
        compiler_params=_SC_PARAMS,
        scratch_types=[
            pltpu.VMEM((CH,), jnp.int32),
            pltpu.VMEM((CH, D), jnp.float32),
            pltpu.SemaphoreType.DMA,
        ],
    )
    return f(stab, pidx)


def _tc_prologue(deg2, x):
    def body(deg_ref, x_ref, dinv_ref, g_ref):
        dinv = lax.rsqrt(jnp.maximum(deg_ref[0, :, 0:1], 1.0))
        dinv_ref[0] = dinv
        g_ref[0] = x_ref[0] * dinv

    return pl.pallas_call(
        body,
        grid=(2, NP // BLK),
        in_specs=[
            pl.BlockSpec((1, BLK, 16), lambda i, j: (i, j, 0)),
            pl.BlockSpec((1, BLK, D), lambda i, j: (i, j, 0)),
        ],
        out_specs=[
            pl.BlockSpec((1, BLK, 1), lambda i, j: (i, j, 0)),
            pl.BlockSpec((1, BLK, D), lambda i, j: (1 - i, j, 0)),
        ],
        out_shape=[
            jax.ShapeDtypeStruct((2, NP, 1), jnp.float32),
            jax.ShapeDtypeStruct((2, NP, D), jnp.float32),
        ],
    )(deg2, x)


def _tc_epilogue(acc, dinv, s_in, relu, need_g):
    def body(acc_ref, dinv_ref, s_ref, s_out_ref, *maybe_g):
        dinv = dinv_ref[0]
        h = acc_ref[0] * dinv
        if relu:
            h = jnp.maximum(h, 0.0)
        s_out_ref[0] = s_ref[0] + h
        if need_g:
            maybe_g[0][0] = h * dinv

    out_specs = [pl.BlockSpec((1, BLK, D), lambda i, j: (i, j, 0))]
    out_shape = [jax.ShapeDtypeStruct((2, NP, D), jnp.float32)]
    if need_g:
        out_specs.append(pl.BlockSpec((1, BLK, D), lambda i, j: (1 - i, j, 0)))
        out_shape.append(jax.ShapeDtypeStruct((2, NP, D), jnp.float32))

    return pl.pallas_call(
        body,
        grid=(2, NP // BLK),
        in_specs=[
            pl.BlockSpec((1, BLK, D), lambda i, j: (i, j, 0)),
            pl.BlockSpec((1, BLK, 1), lambda i, j: (i, j, 0)),
            pl.BlockSpec((1, BLK, D), lambda i, j: (i, j, 0)),
        ],
        out_specs=out_specs,
        out_shape=out_shape,
    )(acc, dinv, s_in)


def _tc_decode(su, si):
    def body(u_ref, i_ref, o_ref):
        o_ref[...] = jnp.sum(
            u_ref[...] * i_ref[...], axis=1, keepdims=True) * (1.0 / 16.0)

    return pl.pallas_call(
        body,
        grid=(BATCH // BLK,),
        in_specs=[
            pl.BlockSpec((BLK, D), lambda i: (i, 0)),
            pl.BlockSpec((BLK, D), lambda i: (i, 0)),
        ],
        out_specs=pl.BlockSpec((BLK, 1), lambda i: (i, 0)),
        out_shape=jax.ShapeDtypeStruct((BATCH, 1), jnp.float32),
    )(su, si)


def kernel(users, items, edge_list, emb_user, emb_item):
    users = users.astype(jnp.int32)
    items = items.astype(jnp.int32)
    el = edge_list.astype(jnp.int32)
    eu = el[:, 0]
    ei = el[:, 1]

    padw = EPT - NE // NS

    def tile_pad(x, pad_tail):
        x = x.reshape(NS, NE // NS)
        return jnp.concatenate([x, pad_tail], axis=1).reshape(-1)

    spread = (jnp.arange(NS * padw, dtype=jnp.int32) % NU).reshape(NS, padw)
    sink = jnp.full((NS, padw), PAD_ROW, jnp.int32)

    gidx = jnp.concatenate([tile_pad(ei, spread), tile_pad(eu + NP, spread + NP)])
    sidx = jnp.concatenate([tile_pad(eu, sink), tile_pad(ei, sink)])

    zdeg = jnp.zeros((NP, 16), jnp.float32)
    zrows = jnp.zeros((NP, D), jnp.float32)
    ones_rows = jnp.ones((CH, 16), jnp.float32)

    deg2 = _sc_degree(sidx, zdeg, ones_rows).reshape(2, NP, 16)

    x = jnp.stack([
        jnp.pad(emb_user, ((0, NP - NU), (0, 0))),
        jnp.pad(emb_item, ((0, NP - NI), (0, 0))),
    ])
    dinv, gtab = _tc_prologue(deg2, x)

    s = x
    for layer in range(NLAYERS):
        acc = _sc_propagate(
            gtab.reshape(2 * NP, D), gidx, sidx, zrows).reshape(2, NP, D)
        if layer < NLAYERS - 1:
            s, gtab = _tc_epilogue(acc, dinv, s, relu=True, need_g=True)
        else:
            (s,) = _tc_epilogue(acc, dinv, s, relu=False, need_g=False)

    pidx = jnp.concatenate([users, items + NP])
    prows = _sc_gather_pairs(s.reshape(2 * NP, D), pidx)
    scores = _tc_decode(prows[:BATCH], prows[BATCH:])
    return scores.reshape(BATCH)


# speedup vs baseline: 28.8928x; 2.1558x over previous
import functools

import jax
import jax.numpy as jnp
from jax import lax
from jax.experimental import pallas as pl
from jax.experimental.pallas import tpu as pltpu
from jax.experimental.pallas import tpu_sc as plsc

NU = 25000
NI = 25000
D = 64
NE = 800000
NLAYERS = 3
BATCH = 4096

NC = 2
NS = 16
NP = 25088
RPT = NP // NS
CH = 128
NCH_PROP = 393
NCH_DEG = 394
NCH_IDX = 396
PAD_ROW = NP - 1
PPT = 2 * BATCH // (NC * NS)

BLK = 512


def _mesh():
    return plsc.VectorSubcoreMesh(
        core_axis_name="c", subcore_axis_name="s",
        num_cores=NC, num_subcores=NS)


_SC_PARAMS = pltpu.CompilerParams(use_tc_tiling_on_sc=False)


def _sc_degree(idxc, zdeg, ones_rows):
    def body(idxc_h, zdeg_h, ones_h, out_h, deg_sp, obuf, iba, ibb, sia, sib):
        c = lax.axis_index("c")
        s = lax.axis_index("s")
        sl = pl.ds(s * RPT, RPT)
        pltpu.sync_copy(zdeg_h.at[sl], deg_sp.at[sl])
        pltpu.sync_copy(ones_h, obuf)
        plsc.subcore_barrier()
        kb = (c * NS + s) * NCH_IDX
        pltpu.async_copy(idxc_h.at[kb], iba, sia)
        pltpu.async_copy(idxc_h.at[kb + 1], ibb, sib)

        def chunk(t, carry):
            j = kb + 2 * t
            pltpu.make_async_copy(idxc_h.at[kb], iba, sia).wait()
            pltpu.sync_copy(obuf, deg_sp.at[iba.at[1]], add=True)
            pltpu.async_copy(idxc_h.at[j + 2], iba, sia)
            pltpu.make_async_copy(idxc_h.at[kb], ibb, sib).wait()
            pltpu.sync_copy(obuf, deg_sp.at[ibb.at[1]], add=True)
            pltpu.async_copy(idxc_h.at[j + 3], ibb, sib)
            return carry

        lax.fori_loop(0, NCH_DEG // 2, chunk, 0)
        pltpu.make_async_copy(idxc_h.at[kb], iba, sia).wait()
        pltpu.make_async_copy(idxc_h.at[kb], ibb, sib).wait()
        plsc.subcore_barrier()
        pltpu.sync_copy(deg_sp.at[sl], out_h.at[pl.ds(c * NP + s * RPT, RPT)])

    f = pl.kernel(
        body,
        out_type=jax.ShapeDtypeStruct((NC * NP, 16), jnp.float32),
        mesh=_mesh(),
        compiler_params=_SC_PARAMS,
        scratch_types=[
            pltpu.VMEM_SHARED((NP, 16), jnp.float32),
            pltpu.VMEM((CH, 16), jnp.float32),
            pltpu.VMEM((2, CH), jnp.int32),
            pltpu.VMEM((2, CH), jnp.int32),
            pltpu.SemaphoreType.DMA,
            pltpu.SemaphoreType.DMA,
        ],
    )
    return f(idxc, zdeg, ones_rows)


def _sc_propagate(gtab, idxc, zrows):
    def body(gtab_h, idxc_h, zrows_h, out_h, acc,
             ib0, ib1, ib2, rw0, rw1, rw2,
             si0, si1, si2, sg0, sg1, sg2):
        ib = [ib0, ib1, ib2]
        rw = [rw0, rw1, rw2]
        si = [si0, si1, si2]
        sg = [sg0, sg1, sg2]
        c = lax.axis_index("c")
        s = lax.axis_index("s")
        sl = pl.ds(s * RPT, RPT)
        pltpu.sync_copy(zrows_h.at[sl], acc.at[sl])
        plsc.subcore_barrier()
        kb = (c * NS + s) * NCH_IDX

        def wait_idx(q):
            pltpu.make_async_copy(idxc_h.at[kb], ib[q], si[q]).wait()

        def wait_gather(q):
            pltpu.make_async_copy(gtab_h.at[ib[q].at[0]], rw[q], sg[q]).wait()

        for q in range(3):
            pltpu.async_copy(idxc_h.at[kb + q], ib[q], si[q])
        for q in range(2):
            wait_idx(q)
            pltpu.async_copy(gtab_h.at[ib[q].at[0]], rw[q], sg[q])

        def chunk(u, carry):
            j = kb + 3 * u
            for q in range(3):
                wait_gather(q)
                pltpu.sync_copy(rw[q], acc.at[ib[q].at[1]], add=True)
                pltpu.async_copy(idxc_h.at[j + q + 3], ib[q], si[q])
                q2 = (q + 2) % 3
                wait_idx(q2)
                pltpu.async_copy(gtab_h.at[ib[q2].at[0]], rw[q2], sg[q2])
            return carry

        lax.fori_loop(0, NCH_PROP // 3, chunk, 0)
        wait_gather(0)
        wait_gather(1)
        wait_idx(2)
        plsc.subcore_barrier()
        pltpu.sync_copy(acc.at[sl], out_h.at[pl.ds(c * NP + s * RPT, RPT)])

    f = pl.kernel(
        body,
        out_type=jax.ShapeDtypeStruct((NC * NP, D), jnp.float32),
        mesh=_mesh(),
        compiler_params=_SC_PARAMS,
        scratch_types=(
            [pltpu.VMEM_SHARED((NP, D), jnp.float32)]
            + [pltpu.VMEM((2, CH), jnp.int32) for _ in range(3)]
            + [pltpu.VMEM((CH, D), jnp.float32) for _ in range(3)]
            + [pltpu.SemaphoreType.DMA for _ in range(6)]
        ),
    )
    return f(gtab, idxc, zrows)


def _sc_gather_pairs(stab, pidx):
    def body(stab_h, pidx_h, out_h, idxp, prow, sem):
        c = lax.axis_index("c")
        s = lax.axis_index("s")
        base = (c * NS + s) * PPT
        for k in range(PPT // CH):
            off = base + k * CH
            pltpu.sync_copy(pidx_h.at[pl.ds(off, CH)], idxp)
            pltpu.async_copy(stab_h.at[idxp], prow, sem).wait()
            pltpu.sync_copy(prow, out_h.at[pl.ds(off, CH)])

    f = pl.kernel(
        body,
        out_type=jax.ShapeDtypeStruct((2 * BATCH, D), jnp.float32),
        mesh=_mesh(),
        compiler_params=_SC_PARAMS,
        scratch_types=[
            pltpu.VMEM((CH,), jnp.int32),
            pltpu.VMEM((CH, D), jnp.float32),
            pltpu.SemaphoreType.DMA,
        ],
    )
    return f(stab, pidx)


def _tc_prologue(deg2, x):
    def body(deg_ref, x_ref, dinv_ref, g_ref):
        dinv = lax.rsqrt(jnp.maximum(deg_ref[0, :, 0:1], 1.0))
        dinv_ref[0] = dinv
        g_ref[0] = x_ref[0] * dinv

    return pl.pallas_call(
        body,
        grid=(2, NP // BLK),
        in_specs=[
            pl.BlockSpec((1, BLK, 16), lambda i, j: (i, j, 0)),
            pl.BlockSpec((1, BLK, D), lambda i, j: (i, j, 0)),
        ],
        out_specs=[
            pl.BlockSpec((1, BLK, 1), lambda i, j: (i, j, 0)),
            pl.BlockSpec((1, BLK, D), lambda i, j: (1 - i, j, 0)),
        ],
        out_shape=[
            jax.ShapeDtypeStruct((2, NP, 1), jnp.float32),
            jax.ShapeDtypeStruct((2, NP, D), jnp.float32),
        ],
    )(deg2, x)


def _tc_epilogue(acc, dinv, s_in, relu, need_g):
    def body(acc_ref, dinv_ref, s_ref, s_out_ref, *maybe_g):
        dinv = dinv_ref[0]
        h = acc_ref[0] * dinv
        if relu:
            h = jnp.maximum(h, 0.0)
        s_out_ref[0] = s_ref[0] + h
        if need_g:
            maybe_g[0][0] = h * dinv

    out_specs = [pl.BlockSpec((1, BLK, D), lambda i, j: (i, j, 0))]
    out_shape = [jax.ShapeDtypeStruct((2, NP, D), jnp.float32)]
    if need_g:
        out_specs.append(pl.BlockSpec((1, BLK, D), lambda i, j: (1 - i, j, 0)))
        out_shape.append(jax.ShapeDtypeStruct((2, NP, D), jnp.float32))

    return pl.pallas_call(
        body,
        grid=(2, NP // BLK),
        in_specs=[
            pl.BlockSpec((1, BLK, D), lambda i, j: (i, j, 0)),
            pl.BlockSpec((1, BLK, 1), lambda i, j: (i, j, 0)),
            pl.BlockSpec((1, BLK, D), lambda i, j: (i, j, 0)),
        ],
        out_specs=out_specs,
        out_shape=out_shape,
    )(acc, dinv, s_in)


def _tc_decode(su, si):
    def body(u_ref, i_ref, o_ref):
        o_ref[...] = jnp.sum(
            u_ref[...] * i_ref[...], axis=1, keepdims=True) * (1.0 / 16.0)

    return pl.pallas_call(
        body,
        grid=(BATCH // BLK,),
        in_specs=[
            pl.BlockSpec((BLK, D), lambda i: (i, 0)),
            pl.BlockSpec((BLK, D), lambda i: (i, 0)),
        ],
        out_specs=pl.BlockSpec((BLK, 1), lambda i: (i, 0)),
        out_shape=jax.ShapeDtypeStruct((BATCH, 1), jnp.float32),
    )(su, si)


def kernel(users, items, edge_list, emb_user, emb_item):
    users = users.astype(jnp.int32)
    items = items.astype(jnp.int32)
    el = edge_list.astype(jnp.int32)
    eu = el[:, 0]
    ei = el[:, 1]

    padw = NCH_IDX * CH - NE // NS

    spread = (jnp.arange(NS * padw, dtype=jnp.int32) % NU).reshape(NS, padw)
    sink = jnp.full((NS, padw), PAD_ROW, jnp.int32)

    def build(gv, sv, gpad):
        g3 = jnp.concatenate(
            [gv.reshape(NS, NE // NS), gpad], axis=1).reshape(NS, NCH_IDX, CH)
        s3 = jnp.concatenate(
            [sv.reshape(NS, NE // NS), sink], axis=1).reshape(NS, NCH_IDX, CH)
        return jnp.stack([g3, s3], axis=2)

    idxc = jnp.concatenate([
        build(ei, eu, spread),
        build(eu + NP, ei, spread + NP),
    ]).reshape(NC * NS * NCH_IDX, 2, CH)

    zdeg = jnp.zeros((NP, 16), jnp.float32)
    zrows = jnp.zeros((NP, D), jnp.float32)
    ones_rows = jnp.ones((CH, 16), jnp.float32)

    deg2 = _sc_degree(idxc, zdeg, ones_rows).reshape(2, NP, 16)

    x = jnp.stack([
        jnp.pad(emb_user, ((0, NP - NU), (0, 0))),
        jnp.pad(emb_item, ((0, NP - NI), (0, 0))),
    ])
    dinv, gtab = _tc_prologue(deg2, x)

    s = x
    for layer in range(NLAYERS):
        acc = _sc_propagate(
            gtab.reshape(2 * NP, D), idxc, zrows).reshape(2, NP, D)
        if layer < NLAYERS - 1:
            s, gtab = _tc_epilogue(acc, dinv, s, relu=True, need_g=True)
        else:
            (s,) = _tc_epilogue(acc, dinv, s, relu=False, need_g=False)

    pidx = jnp.concatenate([users, items + NP])
    prows = _sc_gather_pairs(s.reshape(2 * NP, D), pidx)
    scores = _tc_decode(prows[:BATCH], prows[BATCH:])
    return scores.reshape(BATCH)


# speedup vs baseline: 33.0527x; 1.1440x over previous
import functools

import jax
import jax.numpy as jnp
from jax import lax
from jax.experimental import pallas as pl
from jax.experimental.pallas import tpu as pltpu
from jax.experimental.pallas import tpu_sc as plsc

NU = 25000
NI = 25000
D = 64
NE = 800000
NLAYERS = 3
BATCH = 4096

NC = 2
NS = 16
NP = 25088
RPT = NP // NS
CH = 128
NCH_PROP = 393
NCH_IDX = 396
PAD_ROW = NP - 1
PPT = 2 * BATCH // (NC * NS)

BLK = 3584


def _mesh():
    return plsc.VectorSubcoreMesh(
        core_axis_name="c", subcore_axis_name="s",
        num_cores=NC, num_subcores=NS)


_SC_PARAMS = pltpu.CompilerParams(use_tc_tiling_on_sc=False)


def _sc_degree(idxc, zdeg, ones_rows):
    def body(idxc_h, zdeg_h, ones_h, out_h, deg_sp, obuf,
             ib0, ib1, ib2, si0, si1, si2):
        ib = [ib0, ib1, ib2]
        si = [si0, si1, si2]
        c = lax.axis_index("c")
        s = lax.axis_index("s")
        sl = pl.ds(s * RPT, RPT)
        pltpu.sync_copy(zdeg_h.at[sl], deg_sp.at[sl])
        pltpu.sync_copy(ones_h, obuf)
        plsc.subcore_barrier()
        kb = (c * NS + s) * NCH_IDX

        def wait_idx(q):
            pltpu.make_async_copy(idxc_h.at[kb], ib[q], si[q]).wait()

        for q in range(3):
            pltpu.async_copy(idxc_h.at[kb + q], ib[q], si[q])

        def chunk(u, carry):
            j = kb + 3 * u
            for q in range(3):
                wait_idx(q)
                pltpu.sync_copy(obuf, deg_sp.at[ib[q].at[1]], add=True)
                pltpu.async_copy(idxc_h.at[j + q + 3], ib[q], si[q])
            return carry

        lax.fori_loop(0, NCH_PROP // 3, chunk, 0)
        for q in range(3):
            wait_idx(q)
        plsc.subcore_barrier()
        pltpu.sync_copy(deg_sp.at[sl], out_h.at[pl.ds(c * NP + s * RPT, RPT)])

    f = pl.kernel(
        body,
        out_type=jax.ShapeDtypeStruct((NC * NP, 16), jnp.float32),
        mesh=_mesh(),
        compiler_params=_SC_PARAMS,
        scratch_types=(
            [pltpu.VMEM_SHARED((NP, 16), jnp.float32),
             pltpu.VMEM((CH, 16), jnp.float32)]
            + [pltpu.VMEM((2, CH), jnp.int32) for _ in range(3)]
            + [pltpu.SemaphoreType.DMA for _ in range(3)]
        ),
    )
    return f(idxc, zdeg, ones_rows)


def _sc_propagate(gtab, idxc, zrows):
    def body(gtab_h, idxc_h, zrows_h, out_h, acc,
             ib0, ib1, ib2, rw0, rw1, rw2,
             si0, si1, si2, sg0, sg1, sg2):
        ib = [ib0, ib1, ib2]
        rw = [rw0, rw1, rw2]
        si = [si0, si1, si2]
        sg = [sg0, sg1, sg2]
        c = lax.axis_index("c")
        s = lax.axis_index("s")
        sl = pl.ds(s * RPT, RPT)
        pltpu.sync_copy(zrows_h.at[sl], acc.at[sl])
        plsc.subcore_barrier()
        kb = (c * NS + s) * NCH_IDX

        def wait_idx(q):
            pltpu.make_async_copy(idxc_h.at[kb], ib[q], si[q]).wait()

        def wait_gather(q):
            pltpu.make_async_copy(gtab_h.at[ib[q].at[0]], rw[q], sg[q]).wait()

        for q in range(3):
            pltpu.async_copy(idxc_h.at[kb + q], ib[q], si[q])
        for q in range(2):
            wait_idx(q)
            pltpu.async_copy(gtab_h.at[ib[q].at[0]], rw[q], sg[q])

        def chunk(u, carry):
            j = kb + 3 * u
            for q in range(3):
                wait_gather(q)
                pltpu.sync_copy(rw[q], acc.at[ib[q].at[1]], add=True)
                pltpu.async_copy(idxc_h.at[j + q + 3], ib[q], si[q])
                q2 = (q + 2) % 3
                wait_idx(q2)
                pltpu.async_copy(gtab_h.at[ib[q2].at[0]], rw[q2], sg[q2])
            return carry

        lax.fori_loop(0, NCH_PROP // 3, chunk, 0)
        wait_gather(0)
        wait_gather(1)
        wait_idx(2)
        plsc.subcore_barrier()
        pltpu.sync_copy(acc.at[sl], out_h.at[pl.ds(c * NP + s * RPT, RPT)])

    f = pl.kernel(
        body,
        out_type=jax.ShapeDtypeStruct((NC * NP, D), jnp.float32),
        mesh=_mesh(),
        compiler_params=_SC_PARAMS,
        scratch_types=(
            [pltpu.VMEM_SHARED((NP, D), jnp.float32)]
            + [pltpu.VMEM((2, CH), jnp.int32) for _ in range(3)]
            + [pltpu.VMEM((CH, D), jnp.float32) for _ in range(3)]
            + [pltpu.SemaphoreType.DMA for _ in range(6)]
        ),
    )
    return f(gtab, idxc, zrows)


def _sc_gather_pairs(stab, pidx):
    def body(stab_h, pidx_h, out_h, idxp, prow, sem):
        c = lax.axis_index("c")
        s = lax.axis_index("s")
        base = (c * NS + s) * PPT
        for k in range(PPT // CH):
            off = base + k * CH
            pltpu.sync_copy(pidx_h.at[pl.ds(off, CH)], idxp)
            pltpu.async_copy(stab_h.at[idxp], prow, sem).wait()
            pltpu.sync_copy(prow, out_h.at[pl.ds(off, CH)])

    f = pl.kernel(
        body,
        out_type=jax.ShapeDtypeStruct((2 * BATCH, D), jnp.float32),
        mesh=_mesh(),
        compiler_params=_SC_PARAMS,
        scratch_types=[
            pltpu.VMEM((CH,), jnp.int32),
            pltpu.VMEM((CH, D), jnp.float32),
            pltpu.SemaphoreType.DMA,
        ],
    )
    return f(stab, pidx)


def _tc_prologue(deg2, x):
    nb = 2 * NP // BLK
    swap = nb // 2

    def body(deg_ref, x_ref, dinv_ref, g_ref):
        dinv = lax.rsqrt(jnp.maximum(deg_ref[:, 0:1], 1.0))
        dinv_ref[...] = dinv
        g_ref[...] = x_ref[...] * dinv

    return pl.pallas_call(
        body,
        grid=(nb,),
        in_specs=[
            pl.BlockSpec((BLK, 16), lambda i: (i, 0)),
            pl.BlockSpec((BLK, D), lambda i: (i, 0)),
        ],
        out_specs=[
            pl.BlockSpec((BLK, 1), lambda i: (i, 0)),
            pl.BlockSpec((BLK, D), lambda i: ((i + swap) % nb, 0)),
        ],
        out_shape=[
            jax.ShapeDtypeStruct((2 * NP, 1), jnp.float32),
            jax.ShapeDtypeStruct((2 * NP, D), jnp.float32),
        ],
    )(deg2, x)


def _tc_epilogue(acc, dinv, s_in, relu, need_g):
    nb = 2 * NP // BLK
    swap = nb // 2

    def body(acc_ref, dinv_ref, s_ref, s_out_ref, *maybe_g):
        dinv = dinv_ref[...]
        h = acc_ref[...] * dinv
        if relu:
            h = jnp.maximum(h, 0.0)
        s_out_ref[...] = s_ref[...] + h
        if need_g:
            maybe_g[0][...] = h * dinv

    out_specs = [pl.BlockSpec((BLK, D), lambda i: (i, 0))]
    out_shape = [jax.ShapeDtypeStruct((2 * NP, D), jnp.float32)]
    if need_g:
        out_specs.append(pl.BlockSpec((BLK, D), lambda i: ((i + swap) % nb, 0)))
        out_shape.append(jax.ShapeDtypeStruct((2 * NP, D), jnp.float32))

    return pl.pallas_call(
        body,
        grid=(2 * NP // BLK,),
        in_specs=[
            pl.BlockSpec((BLK, D), lambda i: (i, 0)),
            pl.BlockSpec((BLK, 1), lambda i: (i, 0)),
            pl.BlockSpec((BLK, D), lambda i: (i, 0)),
        ],
        out_specs=out_specs,
        out_shape=out_shape,
    )(acc, dinv, s_in)


def _tc_decode(su, si):
    def body(u_ref, i_ref, o_ref):
        o_ref[...] = jnp.sum(
            u_ref[...] * i_ref[...], axis=1, keepdims=True) * (1.0 / 16.0)

    return pl.pallas_call(
        body,
        grid=(2,),
        in_specs=[
            pl.BlockSpec((BATCH // 2, D), lambda i: (i, 0)),
            pl.BlockSpec((BATCH // 2, D), lambda i: (i, 0)),
        ],
        out_specs=pl.BlockSpec((BATCH // 2, 1), lambda i: (i, 0)),
        out_shape=jax.ShapeDtypeStruct((BATCH, 1), jnp.float32),
    )(su, si)


def kernel(users, items, edge_list, emb_user, emb_item):
    users = users.astype(jnp.int32)
    items = items.astype(jnp.int32)
    el = edge_list.astype(jnp.int32)
    eu = el[:, 0]
    ei = el[:, 1]

    padw = NCH_IDX * CH - NE // NS

    spread = (jnp.arange(NS * padw, dtype=jnp.int32) % NU).reshape(NS, padw)
    sink = jnp.full((NS, padw), PAD_ROW, jnp.int32)

    def build(gv, sv, gpad):
        g3 = jnp.concatenate(
            [gv.reshape(NS, NE // NS), gpad], axis=1).reshape(NS, NCH_IDX, CH)
        s3 = jnp.concatenate(
            [sv.reshape(NS, NE // NS), sink], axis=1).reshape(NS, NCH_IDX, CH)
        return jnp.stack([g3, s3], axis=2)

    idxc = jnp.concatenate([
        build(ei, eu, spread),
        build(eu + NP, ei, spread + NP),
    ]).reshape(NC * NS * NCH_IDX, 2, CH)

    zdeg = jnp.zeros((NP, 16), jnp.float32)
    zrows = jnp.zeros((NP, D), jnp.float32)
    ones_rows = jnp.ones((CH, 16), jnp.float32)

    deg2 = _sc_degree(idxc, zdeg, ones_rows)

    x = jnp.concatenate([
        jnp.pad(emb_user, ((0, NP - NU), (0, 0))),
        jnp.pad(emb_item, ((0, NP - NI), (0, 0))),
    ])
    dinv, gtab = _tc_prologue(deg2, x)

    s = x
    for layer in range(NLAYERS):
        acc = _sc_propagate(gtab, idxc, zrows)
        if layer < NLAYERS - 1:
            s, gtab = _tc_epilogue(acc, dinv, s, relu=True, need_g=True)
        else:
            (s,) = _tc_epilogue(acc, dinv, s, relu=False, need_g=False)

    pidx = jnp.concatenate([users, items + NP])
    prows = _sc_gather_pairs(s, pidx)
    scores = _tc_decode(prows[:BATCH], prows[BATCH:])
    return scores.reshape(BATCH)


# speedup vs baseline: 35.8787x; 1.0855x over previous
import functools

import jax
import jax.numpy as jnp
from jax import lax
from jax.experimental import pallas as pl
from jax.experimental.pallas import tpu as pltpu
from jax.experimental.pallas import tpu_sc as plsc

NU = 25000
NI = 25000
D = 64
NE = 800000
NLAYERS = 3
BATCH = 4096

NC = 2
NS = 16
NP = 25088
RPT = NP // NS
CH = 128
NCH_PROP = 393
NCH_IDX = 396
PAD_ROW = NP - 1
PPT = 2 * BATCH // (NC * NS)

BLK = 3584


def _mesh():
    return plsc.VectorSubcoreMesh(
        core_axis_name="c", subcore_axis_name="s",
        num_cores=NC, num_subcores=NS)


_SC_PARAMS = pltpu.CompilerParams(
    use_tc_tiling_on_sc=False, needs_layout_passes=False)


def _sc_degree(idxc, zdeg, ones_rows):
    def body(idxc_h, zdeg_h, ones_h, out_h, deg_sp, obuf,
             ib0, ib1, ib2, si0, si1, si2):
        ib = [ib0, ib1, ib2]
        si = [si0, si1, si2]
        c = lax.axis_index("c")
        s = lax.axis_index("s")
        sl = pl.ds(s * RPT, RPT)
        pltpu.sync_copy(zdeg_h.at[sl], deg_sp.at[sl])
        pltpu.sync_copy(ones_h, obuf)
        plsc.subcore_barrier()
        kb = (c * NS + s) * NCH_IDX

        def wait_idx(q):
            pltpu.make_async_copy(idxc_h.at[kb], ib[q], si[q]).wait()

        for q in range(3):
            pltpu.async_copy(idxc_h.at[kb + q], ib[q], si[q])

        def chunk(u, carry):
            j = kb + 3 * u
            for q in range(3):
                wait_idx(q)
                pltpu.sync_copy(obuf, deg_sp.at[ib[q].at[1]], add=True)
                pltpu.async_copy(idxc_h.at[j + q + 3], ib[q], si[q])
            return carry

        lax.fori_loop(0, NCH_PROP // 3, chunk, 0)
        for q in range(3):
            wait_idx(q)
        plsc.subcore_barrier()
        pltpu.sync_copy(deg_sp.at[sl], out_h.at[pl.ds(c * NP + s * RPT, RPT)])

    f = pl.kernel(
        body,
        out_type=jax.ShapeDtypeStruct((NC * NP, 16), jnp.float32),
        mesh=_mesh(),
        compiler_params=_SC_PARAMS,
        scratch_types=(
            [pltpu.VMEM_SHARED((NP, 16), jnp.float32),
             pltpu.VMEM((CH, 16), jnp.float32)]
            + [pltpu.VMEM((2, CH), jnp.int32) for _ in range(3)]
            + [pltpu.SemaphoreType.DMA for _ in range(3)]
        ),
    )
    return f(idxc, zdeg, ones_rows)


EPC = 112


def _sc_propagate(gtab, idxc, zrows, dinv, s_in, relu, need_g):
    def body(gtab_h, idxc_h, zrows_h, dinv_h, sin_h, *rest):
        if need_g:
            sout_h, g_h = rest[0], rest[1]
            scratch = rest[2:]
        else:
            sout_h = rest[0]
            g_h = None
            scratch = rest[1:]
        (acc, ib0, ib1, ib2, rw0, rw1, rw2, dv,
         si0, si1, si2, sg0, sg1, sg2) = scratch
        ib = [ib0, ib1, ib2]
        rw = [rw0, rw1, rw2]
        si = [si0, si1, si2]
        sg = [sg0, sg1, sg2]
        c = lax.axis_index("c")
        s = lax.axis_index("s")
        sl = pl.ds(s * RPT, RPT)
        pltpu.sync_copy(zrows_h.at[sl], acc.at[sl])
        plsc.subcore_barrier()
        kb = (c * NS + s) * NCH_IDX

        def wait_idx(q):
            pltpu.make_async_copy(idxc_h.at[kb], ib[q], si[q]).wait()

        def wait_gather(q):
            pltpu.make_async_copy(gtab_h.at[ib[q].at[0]], rw[q], sg[q]).wait()

        for q in range(3):
            pltpu.async_copy(idxc_h.at[kb + q], ib[q], si[q])
        for q in range(2):
            wait_idx(q)
            pltpu.async_copy(gtab_h.at[ib[q].at[0]], rw[q], sg[q])

        def chunk(u, carry):
            j = kb + 3 * u
            for q in range(3):
                wait_gather(q)
                pltpu.sync_copy(rw[q], acc.at[ib[q].at[1]], add=True)
                pltpu.async_copy(idxc_h.at[j + q + 3], ib[q], si[q])
                q2 = (q + 2) % 3
                wait_idx(q2)
                pltpu.async_copy(gtab_h.at[ib[q2].at[0]], rw[q2], sg[q2])
            return carry

        lax.fori_loop(0, NCH_PROP // 3, chunk, 0)
        wait_gather(0)
        wait_gather(1)
        wait_idx(2)
        plsc.subcore_barrier()

        base = c * NP + s * RPT
        gbase = (1 - c) * NP + s * RPT
        pltpu.sync_copy(dinv_h.at[pl.ds(base, RPT)], dv)

        def ep(k, carry):
            r0 = k * EPC
            ea = rw0.at[pl.ds(0, EPC)]
            eb = rw1.at[pl.ds(0, EPC)]
            pltpu.sync_copy(acc.at[pl.ds(s * RPT + r0, EPC)], ea)
            pltpu.sync_copy(sin_h.at[pl.ds(base + r0, EPC)], eb)

            def row(i, carry2):
                dvs = plsc.load_gather(
                    dv, [jnp.full((16,), r0 + i, jnp.int32)])
                for q in range(D // 16):
                    cs = pl.ds(16 * q, 16)
                    h = rw0[i, cs] * dvs
                    if relu:
                        h = jnp.maximum(h, 0.0)
                    rw1[i, cs] = rw1[i, cs] + h
                    if need_g:
                        rw0[i, cs] = h * dvs
                return carry2

            lax.fori_loop(0, EPC, row, 0)
            pltpu.sync_copy(eb, sout_h.at[pl.ds(base + r0, EPC)])
            if need_g:
                pltpu.sync_copy(ea, g_h.at[pl.ds(gbase + r0, EPC)])
            return carry

        lax.fori_loop(0, RPT // EPC, ep, 0)

    out_type = [jax.ShapeDtypeStruct((NC * NP, D), jnp.float32)]
    if need_g:
        out_type.append(jax.ShapeDtypeStruct((NC * NP, D), jnp.float32))
    f = pl.kernel(
        body,
        out_type=out_type,
        mesh=_mesh(),
        compiler_params=_SC_PARAMS,
        scratch_types=(
            [pltpu.VMEM_SHARED((NP, D), jnp.float32)]
            + [pltpu.VMEM((2, CH), jnp.int32) for _ in range(3)]
            + [pltpu.VMEM((CH, D), jnp.float32) for _ in range(3)]
            + [pltpu.VMEM((RPT,), jnp.float32)]
            + [pltpu.SemaphoreType.DMA for _ in range(6)]
        ),
    )
    return f(gtab, idxc, zrows, dinv, s_in)


def _sc_gather_pairs(stab, pidx):
    def body(stab_h, pidx_h, out_h, idxp, prow, sem):
        c = lax.axis_index("c")
        s = lax.axis_index("s")
        base = (c * NS + s) * PPT
        for k in range(PPT // CH):
            off = base + k * CH
            pltpu.sync_copy(pidx_h.at[pl.ds(off, CH)], idxp)
            pltpu.async_copy(stab_h.at[idxp], prow, sem).wait()
            pltpu.sync_copy(prow, out_h.at[pl.ds(off, CH)])

    f = pl.kernel(
        body,
        out_type=jax.ShapeDtypeStruct((2 * BATCH, D), jnp.float32),
        mesh=_mesh(),
        compiler_params=_SC_PARAMS,
        scratch_types=[
            pltpu.VMEM((CH,), jnp.int32),
            pltpu.VMEM((CH, D), jnp.float32),
            pltpu.SemaphoreType.DMA,
        ],
    )
    return f(stab, pidx)


def _tc_prologue(deg2, x):
    nb = 2 * NP // BLK
    swap = nb // 2

    def body(deg_ref, x_ref, dinv_ref, g_ref):
        dinv = lax.rsqrt(jnp.maximum(deg_ref[:, 0:1], 1.0))
        dinv_ref[...] = dinv
        g_ref[...] = x_ref[...] * dinv

    return pl.pallas_call(
        body,
        grid=(nb,),
        in_specs=[
            pl.BlockSpec((BLK, 16), lambda i: (i, 0)),
            pl.BlockSpec((BLK, D), lambda i: (i, 0)),
        ],
        out_specs=[
            pl.BlockSpec((BLK, 1), lambda i: (i, 0)),
            pl.BlockSpec((BLK, D), lambda i: ((i + swap) % nb, 0)),
        ],
        out_shape=[
            jax.ShapeDtypeStruct((2 * NP, 1), jnp.float32),
            jax.ShapeDtypeStruct((2 * NP, D), jnp.float32),
        ],
    )(deg2, x)


def _tc_decode(su, si):
    def body(u_ref, i_ref, o_ref):
        o_ref[...] = jnp.sum(
            u_ref[...] * i_ref[...], axis=1, keepdims=True) * (1.0 / 16.0)

    return pl.pallas_call(
        body,
        grid=(2,),
        in_specs=[
            pl.BlockSpec((BATCH // 2, D), lambda i: (i, 0)),
            pl.BlockSpec((BATCH // 2, D), lambda i: (i, 0)),
        ],
        out_specs=pl.BlockSpec((BATCH // 2, 1), lambda i: (i, 0)),
        out_shape=jax.ShapeDtypeStruct((BATCH, 1), jnp.float32),
    )(su, si)


def kernel(users, items, edge_list, emb_user, emb_item):
    users = users.astype(jnp.int32)
    items = items.astype(jnp.int32)
    el = edge_list.astype(jnp.int32)
    eu = el[:, 0]
    ei = el[:, 1]

    padw = NCH_IDX * CH - NE // NS

    spread = (jnp.arange(NS * padw, dtype=jnp.int32) % NU).reshape(NS, padw)
    sink = jnp.full((NS, padw), PAD_ROW, jnp.int32)

    def build(gv, sv, gpad):
        g3 = jnp.concatenate(
            [gv.reshape(NS, NE // NS), gpad], axis=1).reshape(NS, NCH_IDX, CH)
        s3 = jnp.concatenate(
            [sv.reshape(NS, NE // NS), sink], axis=1).reshape(NS, NCH_IDX, CH)
        return jnp.stack([g3, s3], axis=2)

    idxc = jnp.concatenate([
        build(ei, eu, spread),
        build(eu + NP, ei, spread + NP),
    ]).reshape(NC * NS * NCH_IDX, 2, CH)

    zdeg = jnp.zeros((NP, 16), jnp.float32)
    zrows = jnp.zeros((NP, D), jnp.float32)
    ones_rows = jnp.ones((CH, 16), jnp.float32)

    deg2 = _sc_degree(idxc, zdeg, ones_rows)

    x = jnp.concatenate([
        jnp.pad(emb_user, ((0, NP - NU), (0, 0))),
        jnp.pad(emb_item, ((0, NP - NI), (0, 0))),
    ])
    dinv, gtab = _tc_prologue(deg2, x)
    dinv = dinv.reshape(2 * NP)

    s = x
    for layer in range(NLAYERS):
        if layer < NLAYERS - 1:
            s, gtab = _sc_propagate(
                gtab, idxc, zrows, dinv, s, relu=True, need_g=True)
        else:
            (s,) = _sc_propagate(
                gtab, idxc, zrows, dinv, s, relu=False, need_g=False)

    pidx = jnp.concatenate([users, items + NP])
    prows = _sc_gather_pairs(s, pidx)
    scores = _tc_decode(prows[:BATCH], prows[BATCH:])
    return scores.reshape(BATCH)


# speedup vs baseline: 37.3168x; 1.0401x over previous
import functools

import jax
import jax.numpy as jnp
from jax import lax
from jax.experimental import pallas as pl
from jax.experimental.pallas import tpu as pltpu
from jax.experimental.pallas import tpu_sc as plsc

NU = 25000
NI = 25000
D = 64
NE = 800000
NLAYERS = 3
BATCH = 4096

NC = 2
NS = 16
NP = 25088
RPT = NP // NS
CH = 128
NCH_PROP = 393
NCH_IDX = 396
PAD_ROW = NP - 1
PPT = 2 * BATCH // (NC * NS)

BLK = 3584


def _mesh():
    return plsc.VectorSubcoreMesh(
        core_axis_name="c", subcore_axis_name="s",
        num_cores=NC, num_subcores=NS)


_SC_PARAMS = pltpu.CompilerParams(
    use_tc_tiling_on_sc=False, needs_layout_passes=False)


def _rsqrt16(x):
    y = plsc.bitcast(jnp.int32(0x5F3759DF) - (plsc.bitcast(x, jnp.int32) >> 1),
                     jnp.float32)
    y = y * (1.5 - 0.5 * x * y * y)
    y = y * (1.5 - 0.5 * x * y * y)
    return y


def _sc_degree_prologue(sidx, zdeg, ones_rows, x):
    def body(sidx_h, zdeg_h, ones_h, x_h, dinv_h, g_h, deg_sp, obuf, xb, dvb,
             ib0, ib1, ib2, si0, si1, si2):
        ib = [ib0, ib1, ib2]
        si = [si0, si1, si2]
        c = lax.axis_index("c")
        s = lax.axis_index("s")
        sl = pl.ds(s * RPT, RPT)
        pltpu.sync_copy(zdeg_h.at[sl], deg_sp.at[sl])
        pltpu.sync_copy(ones_h, obuf)
        plsc.subcore_barrier()
        kb = (c * NS + s) * NCH_IDX

        def wait_idx(q):
            pltpu.make_async_copy(sidx_h.at[kb], ib[q], si[q]).wait()

        for q in range(3):
            pltpu.async_copy(sidx_h.at[kb + q], ib[q], si[q])

        def chunk(u, carry):
            j = kb + 3 * u
            for q in range(3):
                wait_idx(q)
                pltpu.sync_copy(obuf, deg_sp.at[ib[q]], add=True)
                pltpu.async_copy(sidx_h.at[j + q + 3], ib[q], si[q])
            return carry

        lax.fori_loop(0, NCH_PROP // 3, chunk, 0)
        for q in range(3):
            wait_idx(q)
        plsc.subcore_barrier()

        base = c * NP + s * RPT
        gbase = (1 - c) * NP + s * RPT
        iota16 = lax.iota(jnp.int32, 16)
        zero16 = jnp.zeros((16,), jnp.int32)

        def ep(k, carry):
            r0 = k * EPC
            db = obuf.at[pl.ds(0, EPC)]
            pltpu.sync_copy(deg_sp.at[pl.ds(s * RPT + r0, EPC)], db)
            pltpu.sync_copy(x_h.at[pl.ds(base + r0, EPC)], xb)

            def grp(t, carry2):
                rr = t * 16
                deg = plsc.load_gather(obuf, [iota16 + rr, zero16])
                dvb[pl.ds(rr, 16)] = _rsqrt16(jnp.maximum(deg, 1.0))
                return carry2

            lax.fori_loop(0, EPC // 16, grp, 0)

            def row(i, carry2):
                dvs = plsc.load_gather(dvb, [jnp.full((16,), i, jnp.int32)])
                for q in range(D // 16):
                    cs = pl.ds(16 * q, 16)
                    xb[i, cs] = xb[i, cs] * dvs
                return carry2

            lax.fori_loop(0, EPC, row, 0)
            pltpu.sync_copy(dvb, dinv_h.at[pl.ds(base + r0, EPC)])
            pltpu.sync_copy(xb, g_h.at[pl.ds(gbase + r0, EPC)])
            return carry

        lax.fori_loop(0, RPT // EPC, ep, 0)

    f = pl.kernel(
        body,
        out_type=[
            jax.ShapeDtypeStruct((NC * NP,), jnp.float32),
            jax.ShapeDtypeStruct((NC * NP, D), jnp.float32),
        ],
        mesh=_mesh(),
        compiler_params=_SC_PARAMS,
        scratch_types=(
            [pltpu.VMEM_SHARED((NP, 16), jnp.float32),
             pltpu.VMEM((CH, 16), jnp.float32),
             pltpu.VMEM((EPC, D), jnp.float32),
             pltpu.VMEM((EPC,), jnp.float32)]
            + [pltpu.VMEM((CH,), jnp.int32) for _ in range(3)]
            + [pltpu.SemaphoreType.DMA for _ in range(3)]
        ),
    )
    return f(sidx, zdeg, ones_rows, x)


EPC = 112


def _sc_propagate(gtab, gidx, sidx, zrows, dinv, s_in, relu, need_g):
    def body(gtab_h, gidx_h, sidx_h, zrows_h, dinv_h, sin_h, *rest):
        if need_g:
            sout_h, g_h = rest[0], rest[1]
            scratch = rest[2:]
        else:
            sout_h = rest[0]
            g_h = None
            scratch = rest[1:]
        (acc, ib0, ib1, ib2, rw0, rw1, rw2, dv,
         si0, si1, si2, sg0, sg1, sg2) = scratch
        ib = [ib0, ib1, ib2]
        rw = [rw0, rw1, rw2]
        si = [si0, si1, si2]
        sg = [sg0, sg1, sg2]
        c = lax.axis_index("c")
        s = lax.axis_index("s")
        sl = pl.ds(s * RPT, RPT)
        pltpu.sync_copy(zrows_h.at[sl], acc.at[sl])
        plsc.subcore_barrier()
        kb = (c * NS + s) * NCH_IDX

        def issue_idx(k, q):
            pltpu.async_copy(gidx_h.at[k], ib[q].at[0], si[q])
            pltpu.async_copy(sidx_h.at[k], ib[q].at[1], si[q])

        def wait_idx(q):
            pltpu.make_async_copy(gidx_h.at[pl.ds(kb, 2)], ib[q], si[q]).wait()

        def wait_gather(q):
            pltpu.make_async_copy(gtab_h.at[ib[q].at[0]], rw[q], sg[q]).wait()

        for q in range(3):
            issue_idx(kb + q, q)
        for q in range(2):
            wait_idx(q)
            pltpu.async_copy(gtab_h.at[ib[q].at[0]], rw[q], sg[q])

        def chunk(u, carry):
            j = kb + 3 * u
            for q in range(3):
                wait_gather(q)
                pltpu.sync_copy(rw[q], acc.at[ib[q].at[1]], add=True)
                issue_idx(j + q + 3, q)
                q2 = (q + 2) % 3
                wait_idx(q2)
                pltpu.async_copy(gtab_h.at[ib[q2].at[0]], rw[q2], sg[q2])
            return carry

        lax.fori_loop(0, NCH_PROP // 3, chunk, 0)
        wait_gather(0)
        wait_gather(1)
        wait_idx(2)
        plsc.subcore_barrier()

        base = c * NP + s * RPT
        gbase = (1 - c) * NP + s * RPT
        pltpu.sync_copy(dinv_h.at[pl.ds(base, RPT)], dv)

        def ep(k, carry):
            r0 = k * EPC
            ea = rw0.at[pl.ds(0, EPC)]
            eb = rw1.at[pl.ds(0, EPC)]
            pltpu.sync_copy(acc.at[pl.ds(s * RPT + r0, EPC)], ea)
            pltpu.sync_copy(sin_h.at[pl.ds(base + r0, EPC)], eb)

            def row(i, carry2):
                dvs = plsc.load_gather(
                    dv, [jnp.full((16,), r0 + i, jnp.int32)])
                for q in range(D // 16):
                    cs = pl.ds(16 * q, 16)
                    h = rw0[i, cs] * dvs
                    if relu:
                        h = jnp.maximum(h, 0.0)
                    rw1[i, cs] = rw1[i, cs] + h
                    if need_g:
                        rw0[i, cs] = h * dvs
                return carry2

            lax.fori_loop(0, EPC, row, 0)
            pltpu.sync_copy(eb, sout_h.at[pl.ds(base + r0, EPC)])
            if need_g:
                pltpu.sync_copy(ea, g_h.at[pl.ds(gbase + r0, EPC)])
            return carry

        lax.fori_loop(0, RPT // EPC, ep, 0)

    out_type = [jax.ShapeDtypeStruct((NC * NP, D), jnp.float32)]
    if need_g:
        out_type.append(jax.ShapeDtypeStruct((NC * NP, D), jnp.float32))
    f = pl.kernel(
        body,
        out_type=out_type,
        mesh=_mesh(),
        compiler_params=_SC_PARAMS,
        scratch_types=(
            [pltpu.VMEM_SHARED((NP, D), jnp.float32)]
            + [pltpu.VMEM((2, CH), jnp.int32) for _ in range(3)]
            + [pltpu.VMEM((CH, D), jnp.float32) for _ in range(3)]
            + [pltpu.VMEM((RPT,), jnp.float32)]
            + [pltpu.SemaphoreType.DMA for _ in range(6)]
        ),
    )
    return f(gtab, gidx, sidx, zrows, dinv, s_in)


def _sc_gather_pairs(stab, pidx):
    def body(stab_h, pidx_h, out_h, idxp, prow, sem):
        c = lax.axis_index("c")
        s = lax.axis_index("s")
        base = (c * NS + s) * PPT
        for k in range(PPT // CH):
            off = base + k * CH
            pltpu.sync_copy(pidx_h.at[pl.ds(off, CH)], idxp)
            pltpu.async_copy(stab_h.at[idxp], prow, sem).wait()
            pltpu.sync_copy(prow, out_h.at[pl.ds(off, CH)])

    f = pl.kernel(
        body,
        out_type=jax.ShapeDtypeStruct((2 * BATCH, D), jnp.float32),
        mesh=_mesh(),
        compiler_params=_SC_PARAMS,
        scratch_types=[
            pltpu.VMEM((CH,), jnp.int32),
            pltpu.VMEM((CH, D), jnp.float32),
            pltpu.SemaphoreType.DMA,
        ],
    )
    return f(stab, pidx)


def _tc_decode(su, si):
    def body(u_ref, i_ref, o_ref):
        o_ref[...] = jnp.sum(
            u_ref[...] * i_ref[...], axis=1, keepdims=True) * (1.0 / 16.0)

    return pl.pallas_call(
        body,
        grid=(2,),
        in_specs=[
            pl.BlockSpec((BATCH // 2, D), lambda i: (i, 0)),
            pl.BlockSpec((BATCH // 2, D), lambda i: (i, 0)),
        ],
        out_specs=pl.BlockSpec((BATCH // 2, 1), lambda i: (i, 0)),
        out_shape=jax.ShapeDtypeStruct((BATCH, 1), jnp.float32),
    )(su, si)


def kernel(users, items, edge_list, emb_user, emb_item):
    users = users.astype(jnp.int32)
    items = items.astype(jnp.int32)
    el = edge_list.astype(jnp.int32)
    eu = el[:, 0]
    ei = el[:, 1]

    padw = NCH_IDX * CH - NE // NS

    spread = (jnp.arange(NS * padw, dtype=jnp.int32) % NU).reshape(NS, padw)
    sink = jnp.full((NS, padw), PAD_ROW, jnp.int32)

    def lanes(v, pad):
        return jnp.concatenate(
            [v.reshape(NS, NE // NS), pad], axis=1).reshape(NS * NCH_IDX, CH)

    gidx = jnp.concatenate([lanes(ei, spread), lanes(eu + NP, spread + NP)])
    sidx = jnp.concatenate([lanes(eu, sink), lanes(ei, sink)])

    zdeg = jnp.zeros((NP, 16), jnp.float32)
    zrows = jnp.zeros((NP, D), jnp.float32)
    ones_rows = jnp.ones((CH, 16), jnp.float32)

    x = jnp.concatenate([
        jnp.pad(emb_user, ((0, NP - NU), (0, 0))),
        jnp.pad(emb_item, ((0, NP - NI), (0, 0))),
    ])
    dinv, gtab = _sc_degree_prologue(sidx, zdeg, ones_rows, x)

    s = x
    for layer in range(NLAYERS):
        if layer < NLAYERS - 1:
            s, gtab = _sc_propagate(
                gtab, gidx, sidx, zrows, dinv, s, relu=True, need_g=True)
        else:
            (s,) = _sc_propagate(
                gtab, gidx, sidx, zrows, dinv, s, relu=False, need_g=False)

    pidx = jnp.concatenate([users, items + NP])
    prows = _sc_gather_pairs(s, pidx)
    scores = _tc_decode(prows[:BATCH], prows[BATCH:])
    return scores.reshape(BATCH)


# speedup vs baseline: 37.3762x; 1.0016x over previous
import functools

import jax
import jax.numpy as jnp
from jax import lax
from jax.experimental import pallas as pl
from jax.experimental.pallas import tpu as pltpu
from jax.experimental.pallas import tpu_sc as plsc

NU = 25000
NI = 25000
D = 64
NE = 800000
NLAYERS = 3
BATCH = 4096

NC = 2
NS = 16
NP = 25088
RPT = NP // NS
CH = 128
NCH_PROP = 393
NCH_IDX = 396
PAD_ROW = NP - 1
PPT = 2 * BATCH // (NC * NS)

BLK = 3584


def _mesh():
    return plsc.VectorSubcoreMesh(
        core_axis_name="c", subcore_axis_name="s",
        num_cores=NC, num_subcores=NS)


_SC_PARAMS = pltpu.CompilerParams(
    use_tc_tiling_on_sc=False, needs_layout_passes=False)


def _rsqrt16(x):
    y = plsc.bitcast(jnp.int32(0x5F3759DF) - (plsc.bitcast(x, jnp.int32) >> 1),
                     jnp.float32)
    y = y * (1.5 - 0.5 * x * y * y)
    y = y * (1.5 - 0.5 * x * y * y)
    return y


def _sc_degree_prologue(sidx, zdeg, ones_rows, x):
    def body(sidx_h, zdeg_h, ones_h, x_h, dinv_h, g_h, deg_sp, obuf, xb, dvb,
             ib0, ib1, ib2, si0, si1, si2):
        ib = [ib0, ib1, ib2]
        si = [si0, si1, si2]
        c = lax.axis_index("c")
        s = lax.axis_index("s")
        sl = pl.ds(s * RPT, RPT)
        pltpu.sync_copy(zdeg_h.at[sl], deg_sp.at[sl])
        pltpu.sync_copy(ones_h, obuf)
        plsc.subcore_barrier()
        kb = (c * NS + s) * NCH_IDX

        def wait_idx(q):
            pltpu.make_async_copy(sidx_h.at[kb], ib[q], si[q]).wait()

        for q in range(3):
            pltpu.async_copy(sidx_h.at[kb + q], ib[q], si[q])

        def chunk(u, carry):
            j = kb + 3 * u
            for q in range(3):
                wait_idx(q)
                pltpu.sync_copy(obuf, deg_sp.at[ib[q]], add=True)
                pltpu.async_copy(sidx_h.at[j + q + 3], ib[q], si[q])
            return carry

        lax.fori_loop(0, NCH_PROP // 3, chunk, 0)
        for q in range(3):
            wait_idx(q)
        plsc.subcore_barrier()

        base = c * NP + s * RPT
        gbase = (1 - c) * NP + s * RPT
        iota16 = lax.iota(jnp.int32, 16)
        zero16 = jnp.zeros((16,), jnp.int32)

        def ep(k, carry):
            r0 = k * EPC
            db = obuf.at[pl.ds(0, EPC)]

            @pl.when(k > 0)
            def _():
                pltpu.make_async_copy(
                    dvb, dinv_h.at[pl.ds(base, EPC)], si0).wait()
                pltpu.make_async_copy(
                    xb, g_h.at[pl.ds(gbase, EPC)], si0).wait()

            pltpu.sync_copy(deg_sp.at[pl.ds(s * RPT + r0, EPC)], db)
            pltpu.sync_copy(x_h.at[pl.ds(base + r0, EPC)], xb)

            def grp(t, carry2):
                rr = t * 16
                deg = plsc.load_gather(obuf, [iota16 + rr, zero16])
                dvb[pl.ds(rr, 16)] = _rsqrt16(jnp.maximum(deg, 1.0))
                return carry2

            lax.fori_loop(0, EPC // 16, grp, 0)

            def row(i, carry2):
                dvs = plsc.load_gather(dvb, [jnp.full((16,), i, jnp.int32)])
                for q in range(D // 16):
                    cs = pl.ds(16 * q, 16)
                    xb[i, cs] = xb[i, cs] * dvs
                return carry2

            lax.fori_loop(0, EPC, row, 0)
            pltpu.async_copy(dvb, dinv_h.at[pl.ds(base + r0, EPC)], si0)
            pltpu.async_copy(xb, g_h.at[pl.ds(gbase + r0, EPC)], si0)
            return carry

        lax.fori_loop(0, RPT // EPC, ep, 0)
        pltpu.make_async_copy(dvb, dinv_h.at[pl.ds(base, EPC)], si0).wait()
        pltpu.make_async_copy(xb, g_h.at[pl.ds(gbase, EPC)], si0).wait()

    f = pl.kernel(
        body,
        out_type=[
            jax.ShapeDtypeStruct((NC * NP,), jnp.float32),
            jax.ShapeDtypeStruct((NC * NP, D), jnp.float32),
        ],
        mesh=_mesh(),
        compiler_params=_SC_PARAMS,
        scratch_types=(
            [pltpu.VMEM_SHARED((NP, 16), jnp.float32),
             pltpu.VMEM((CH, 16), jnp.float32),
             pltpu.VMEM((EPC, D), jnp.float32),
             pltpu.VMEM((EPC,), jnp.float32)]
            + [pltpu.VMEM((CH,), jnp.int32) for _ in range(3)]
            + [pltpu.SemaphoreType.DMA for _ in range(3)]
        ),
    )
    return f(sidx, zdeg, ones_rows, x)


EPC = 112


def _sc_propagate(gtab, gidx, sidx, zrows, dinv, s_in, relu, need_g):
    def body(gtab_h, gidx_h, sidx_h, zrows_h, dinv_h, sin_h, *rest):
        if need_g:
            sout_h, g_h = rest[0], rest[1]
            scratch = rest[2:]
        else:
            sout_h = rest[0]
            g_h = None
            scratch = rest[1:]
        (acc, ib0, ib1, ib2, rw0, rw1, rw2, dv,
         si0, si1, si2, sg0, sg1, sg2) = scratch
        ib = [ib0, ib1, ib2]
        rw = [rw0, rw1, rw2]
        si = [si0, si1, si2]
        sg = [sg0, sg1, sg2]
        c = lax.axis_index("c")
        s = lax.axis_index("s")
        sl = pl.ds(s * RPT, RPT)
        pltpu.sync_copy(zrows_h.at[sl], acc.at[sl])
        plsc.subcore_barrier()
        kb = (c * NS + s) * NCH_IDX

        def issue_idx(k, q):
            pltpu.async_copy(gidx_h.at[k], ib[q].at[0], si[q])
            pltpu.async_copy(sidx_h.at[k], ib[q].at[1], si[q])

        def wait_idx(q):
            pltpu.make_async_copy(gidx_h.at[pl.ds(kb, 2)], ib[q], si[q]).wait()

        def wait_gather(q):
            pltpu.make_async_copy(gtab_h.at[ib[q].at[0]], rw[q], sg[q]).wait()

        for q in range(3):
            issue_idx(kb + q, q)
        for q in range(2):
            wait_idx(q)
            pltpu.async_copy(gtab_h.at[ib[q].at[0]], rw[q], sg[q])

        def chunk(u, carry):
            j = kb + 3 * u
            for q in range(3):
                wait_gather(q)
                pltpu.sync_copy(rw[q], acc.at[ib[q].at[1]], add=True)
                issue_idx(j + q + 3, q)
                q2 = (q + 2) % 3
                wait_idx(q2)
                pltpu.async_copy(gtab_h.at[ib[q2].at[0]], rw[q2], sg[q2])
            return carry

        lax.fori_loop(0, NCH_PROP // 3, chunk, 0)
        wait_gather(0)
        wait_gather(1)
        wait_idx(2)
        plsc.subcore_barrier()

        base = c * NP + s * RPT
        gbase = (1 - c) * NP + s * RPT
        pltpu.sync_copy(dinv_h.at[pl.ds(base, RPT)], dv)

        nw = 2 if need_g else 1

        def ep(k, carry):
            r0 = k * EPC
            ea = rw0.at[pl.ds(0, EPC)]
            eb = rw1.at[pl.ds(0, EPC)]

            @pl.when(k > 0)
            def _():
                for _ in range(nw):
                    pltpu.make_async_copy(
                        eb, sout_h.at[pl.ds(base, EPC)], sg0).wait()

            pltpu.sync_copy(acc.at[pl.ds(s * RPT + r0, EPC)], ea)
            pltpu.sync_copy(sin_h.at[pl.ds(base + r0, EPC)], eb)

            def row(i, carry2):
                dvs = plsc.load_gather(
                    dv, [jnp.full((16,), r0 + i, jnp.int32)])
                for q in range(D // 16):
                    cs = pl.ds(16 * q, 16)
                    h = rw0[i, cs] * dvs
                    if relu:
                        h = jnp.maximum(h, 0.0)
                    rw1[i, cs] = rw1[i, cs] + h
                    if need_g:
                        rw0[i, cs] = h * dvs
                return carry2

            lax.fori_loop(0, EPC, row, 0)
            pltpu.async_copy(eb, sout_h.at[pl.ds(base + r0, EPC)], sg0)
            if need_g:
                pltpu.async_copy(ea, g_h.at[pl.ds(gbase + r0, EPC)], sg0)
            return carry

        lax.fori_loop(0, RPT // EPC, ep, 0)
        for _ in range(nw):
            pltpu.make_async_copy(
                rw1.at[pl.ds(0, EPC)], sout_h.at[pl.ds(base, EPC)], sg0).wait()

    out_type = [jax.ShapeDtypeStruct((NC * NP, D), jnp.float32)]
    if need_g:
        out_type.append(jax.ShapeDtypeStruct((NC * NP, D), jnp.float32))
    f = pl.kernel(
        body,
        out_type=out_type,
        mesh=_mesh(),
        compiler_params=_SC_PARAMS,
        scratch_types=(
            [pltpu.VMEM_SHARED((NP, D), jnp.float32)]
            + [pltpu.VMEM((2, CH), jnp.int32) for _ in range(3)]
            + [pltpu.VMEM((CH, D), jnp.float32) for _ in range(3)]
            + [pltpu.VMEM((RPT,), jnp.float32)]
            + [pltpu.SemaphoreType.DMA for _ in range(6)]
        ),
    )
    return f(gtab, gidx, sidx, zrows, dinv, s_in)


def _sc_gather_pairs(stab, pidx):
    def body(stab_h, pidx_h, out_h, idxp, prow, sem):
        c = lax.axis_index("c")
        s = lax.axis_index("s")
        base = (c * NS + s) * PPT
        for k in range(PPT // CH):
            off = base + k * CH
            pltpu.sync_copy(pidx_h.at[pl.ds(off, CH)], idxp)
            pltpu.async_copy(stab_h.at[idxp], prow, sem).wait()
            pltpu.sync_copy(prow, out_h.at[pl.ds(off, CH)])

    f = pl.kernel(
        body,
        out_type=jax.ShapeDtypeStruct((2 * BATCH, D), jnp.float32),
        mesh=_mesh(),
        compiler_params=_SC_PARAMS,
        scratch_types=[
            pltpu.VMEM((CH,), jnp.int32),
            pltpu.VMEM((CH, D), jnp.float32),
            pltpu.SemaphoreType.DMA,
        ],
    )
    return f(stab, pidx)


def _tc_decode(su, si):
    def body(u_ref, i_ref, o_ref):
        o_ref[...] = jnp.sum(
            u_ref[...] * i_ref[...], axis=1, keepdims=True) * (1.0 / 16.0)

    return pl.pallas_call(
        body,
        grid=(2,),
        in_specs=[
            pl.BlockSpec((BATCH // 2, D), lambda i: (i, 0)),
            pl.BlockSpec((BATCH // 2, D), lambda i: (i, 0)),
        ],
        out_specs=pl.BlockSpec((BATCH // 2, 1), lambda i: (i, 0)),
        out_shape=jax.ShapeDtypeStruct((BATCH, 1), jnp.float32),
    )(su, si)


def kernel(users, items, edge_list, emb_user, emb_item):
    users = users.astype(jnp.int32)
    items = items.astype(jnp.int32)
    el = edge_list.astype(jnp.int32)
    eu = el[:, 0]
    ei = el[:, 1]

    padw = NCH_IDX * CH - NE // NS

    spread = (jnp.arange(NS * padw, dtype=jnp.int32) % NU).reshape(NS, padw)
    sink = jnp.full((NS, padw), PAD_ROW, jnp.int32)

    def lanes(v, pad):
        return jnp.concatenate(
            [v.reshape(NS, NE // NS), pad], axis=1).reshape(NS * NCH_IDX, CH)

    gidx = jnp.concatenate([lanes(ei, spread), lanes(eu + NP, spread + NP)])
    sidx = jnp.concatenate([lanes(eu, sink), lanes(ei, sink)])

    zdeg = jnp.zeros((NP, 16), jnp.float32)
    zrows = jnp.zeros((NP, D), jnp.float32)
    ones_rows = jnp.ones((CH, 16), jnp.float32)

    x = jnp.concatenate([
        jnp.pad(emb_user, ((0, NP - NU), (0, 0))),
        jnp.pad(emb_item, ((0, NP - NI), (0, 0))),
    ])
    dinv, gtab = _sc_degree_prologue(sidx, zdeg, ones_rows, x)

    s = x
    for layer in range(NLAYERS):
        if layer < NLAYERS - 1:
            s, gtab = _sc_propagate(
                gtab, gidx, sidx, zrows, dinv, s, relu=True, need_g=True)
        else:
            (s,) = _sc_propagate(
                gtab, gidx, sidx, zrows, dinv, s, relu=False, need_g=False)

    pidx = jnp.concatenate([users, items + NP])
    prows = _sc_gather_pairs(s, pidx)
    scores = _tc_decode(prows[:BATCH], prows[BATCH:])
    return scores.reshape(BATCH)


# speedup vs baseline: 37.3994x; 1.0006x over previous
import functools

import jax
import jax.numpy as jnp
from jax import lax
from jax.experimental import pallas as pl
from jax.experimental.pallas import tpu as pltpu
from jax.experimental.pallas import tpu_sc as plsc

NU = 25000
NI = 25000
D = 64
NE = 800000
NLAYERS = 3
BATCH = 4096

NC = 2
NS = 16
NP = 25088
RPT = NP // NS
CH = 128
NCH_PROP = 393
NCH_IDX = 396
PAD_ROW = NP - 1
PPT = 2 * BATCH // (NC * NS)

BLK = 3584


def _mesh():
    return plsc.VectorSubcoreMesh(
        core_axis_name="c", subcore_axis_name="s",
        num_cores=NC, num_subcores=NS)


_SC_PARAMS = pltpu.CompilerParams(
    use_tc_tiling_on_sc=False, needs_layout_passes=False)


def _rsqrt16(x):
    y = plsc.bitcast(jnp.int32(0x5F3759DF) - (plsc.bitcast(x, jnp.int32) >> 1),
                     jnp.float32)
    y = y * (1.5 - 0.5 * x * y * y)
    y = y * (1.5 - 0.5 * x * y * y)
    return y


def _sc_degree_prologue(sidx, zdeg, ones_rows, x):
    def body(sidx_h, zdeg_h, ones_h, x_h, dinv_h, g_h, deg_sp, obuf, xb, dvb,
             ib0, ib1, ib2, si0, si1, si2):
        ib = [ib0, ib1, ib2]
        si = [si0, si1, si2]
        c = lax.axis_index("c")
        s = lax.axis_index("s")
        sl = pl.ds(s * RPT, RPT)
        pltpu.sync_copy(zdeg_h.at[sl], deg_sp.at[sl])
        pltpu.sync_copy(ones_h, obuf)
        plsc.subcore_barrier()
        kb = (c * NS + s) * NCH_IDX

        def wait_idx(q):
            pltpu.make_async_copy(sidx_h.at[kb], ib[q], si[q]).wait()

        for q in range(3):
            pltpu.async_copy(sidx_h.at[kb + q], ib[q], si[q])

        def chunk(u, carry):
            j = kb + 3 * u
            for q in range(3):
                wait_idx(q)
                pltpu.sync_copy(obuf, deg_sp.at[ib[q]], add=True)
                pltpu.async_copy(sidx_h.at[j + q + 3], ib[q], si[q])
            return carry

        lax.fori_loop(0, NCH_PROP // 3, chunk, 0)
        for q in range(3):
            wait_idx(q)
        plsc.subcore_barrier()

        base = c * NP + s * RPT
        gbase = (1 - c) * NP + s * RPT
        iota16 = lax.iota(jnp.int32, 16)
        zero16 = jnp.zeros((16,), jnp.int32)

        def ep(k, carry):
            r0 = k * EPC
            db = obuf.at[pl.ds(0, EPC)]

            @pl.when(k > 0)
            def _():
                pltpu.make_async_copy(
                    dvb, dinv_h.at[pl.ds(base, EPC)], si0).wait()
                pltpu.make_async_copy(
                    xb, g_h.at[pl.ds(gbase, EPC)], si0).wait()

            pltpu.sync_copy(deg_sp.at[pl.ds(s * RPT + r0, EPC)], db)
            pltpu.sync_copy(x_h.at[pl.ds(base + r0, EPC)], xb)

            def grp(t, carry2):
                rr = t * 16
                deg = plsc.load_gather(obuf, [iota16 + rr, zero16])
                dvb[pl.ds(rr, 16)] = _rsqrt16(jnp.maximum(deg, 1.0))
                return carry2

            lax.fori_loop(0, EPC // 16, grp, 0)

            def row(i, carry2):
                dvs = plsc.load_gather(dvb, [jnp.full((16,), i, jnp.int32)])
                for q in range(D // 16):
                    cs = pl.ds(16 * q, 16)
                    xb[i, cs] = xb[i, cs] * dvs
                return carry2

            lax.fori_loop(0, EPC, row, 0)
            pltpu.async_copy(dvb, dinv_h.at[pl.ds(base + r0, EPC)], si0)
            pltpu.async_copy(xb, g_h.at[pl.ds(gbase + r0, EPC)], si0)
            return carry

        lax.fori_loop(0, RPT // EPC, ep, 0)
        pltpu.make_async_copy(dvb, dinv_h.at[pl.ds(base, EPC)], si0).wait()
        pltpu.make_async_copy(xb, g_h.at[pl.ds(gbase, EPC)], si0).wait()

    f = pl.kernel(
        body,
        out_type=[
            jax.ShapeDtypeStruct((NC * NP,), jnp.float32),
            jax.ShapeDtypeStruct((NC * NP, D), jnp.float32),
        ],
        mesh=_mesh(),
        compiler_params=_SC_PARAMS,
        scratch_types=(
            [pltpu.VMEM_SHARED((NP, 16), jnp.float32),
             pltpu.VMEM((CH, 16), jnp.float32),
             pltpu.VMEM((EPC, D), jnp.float32),
             pltpu.VMEM((EPC,), jnp.float32)]
            + [pltpu.VMEM((CH,), jnp.int32) for _ in range(3)]
            + [pltpu.SemaphoreType.DMA for _ in range(3)]
        ),
    )
    return f(sidx, zdeg, ones_rows, x)


EPC = 112


def _sc_propagate(gtab, gidx, sidx, zrows, dinv, s_in, relu, need_g):
    def body(gtab_h, gidx_h, sidx_h, zrows_h, dinv_h, sin_h, *rest):
        if need_g:
            sout_h, g_h = rest[0], rest[1]
            scratch = rest[2:]
        else:
            sout_h = rest[0]
            g_h = None
            scratch = rest[1:]
        (acc, ib0, ib1, ib2, rw0, rw1, rw2, dv,
         si0, si1, si2, sg0, sg1, sg2) = scratch
        ib = [ib0, ib1, ib2]
        rw = [rw0, rw1, rw2]
        si = [si0, si1, si2]
        sg = [sg0, sg1, sg2]
        c = lax.axis_index("c")
        s = lax.axis_index("s")
        sl = pl.ds(s * RPT, RPT)
        pltpu.sync_copy(zrows_h.at[sl], acc.at[sl])
        plsc.subcore_barrier()
        kb = (c * NS + s) * NCH_IDX

        def issue_idx(k, q):
            pltpu.async_copy(gidx_h.at[k], ib[q].at[0], si[q])
            pltpu.async_copy(sidx_h.at[k], ib[q].at[1], si[q])

        def wait_idx(q):
            pltpu.make_async_copy(gidx_h.at[pl.ds(kb, 2)], ib[q], si[q]).wait()

        def wait_gather(q):
            pltpu.make_async_copy(gtab_h.at[ib[q].at[0]], rw[q], sg[q]).wait()

        for q in range(3):
            issue_idx(kb + q, q)
        for q in range(2):
            wait_idx(q)
            pltpu.async_copy(gtab_h.at[ib[q].at[0]], rw[q], sg[q])

        def chunk(u, carry):
            j = kb + 3 * u
            for q in range(3):
                wait_gather(q)
                pltpu.sync_copy(rw[q], acc.at[ib[q].at[1]], add=True)
                issue_idx(j + q + 3, q)
                q2 = (q + 2) % 3
                wait_idx(q2)
                pltpu.async_copy(gtab_h.at[ib[q2].at[0]], rw[q2], sg[q2])
            return carry

        lax.fori_loop(0, NCH_PROP // 3, chunk, 0)
        wait_gather(0)
        wait_gather(1)
        wait_idx(2)
        plsc.subcore_barrier()

        base = c * NP + s * RPT
        gbase = (1 - c) * NP + s * RPT
        pltpu.sync_copy(dinv_h.at[pl.ds(base, RPT)], dv)

        nw = 2 if need_g else 1

        def ep(k, carry):
            r0 = k * EPC
            ea = rw0.at[pl.ds(0, EPC)]
            eb = rw1.at[pl.ds(0, EPC)]

            @pl.when(k > 0)
            def _():
                for _ in range(nw):
                    pltpu.make_async_copy(
                        eb, sout_h.at[pl.ds(base, EPC)], sg0).wait()

            pltpu.sync_copy(acc.at[pl.ds(s * RPT + r0, EPC)], ea)
            pltpu.sync_copy(sin_h.at[pl.ds(base + r0, EPC)], eb)

            def row(i, carry2):
                dvs = plsc.load_gather(
                    dv, [jnp.full((16,), r0 + i, jnp.int32)])
                for q in range(D // 16):
                    cs = pl.ds(16 * q, 16)
                    h = rw0[i, cs] * dvs
                    if relu:
                        h = jnp.maximum(h, 0.0)
                    rw1[i, cs] = rw1[i, cs] + h
                    if need_g:
                        rw0[i, cs] = h * dvs
                return carry2

            lax.fori_loop(0, EPC, row, 0)
            pltpu.async_copy(eb, sout_h.at[pl.ds(base + r0, EPC)], sg0)
            if need_g:
                pltpu.async_copy(ea, g_h.at[pl.ds(gbase + r0, EPC)], sg0)
            return carry

        lax.fori_loop(0, RPT // EPC, ep, 0)
        for _ in range(nw):
            pltpu.make_async_copy(
                rw1.at[pl.ds(0, EPC)], sout_h.at[pl.ds(base, EPC)], sg0).wait()

    out_type = [jax.ShapeDtypeStruct((NC * NP, D), jnp.float32)]
    if need_g:
        out_type.append(jax.ShapeDtypeStruct((NC * NP, D), jnp.float32))
    f = pl.kernel(
        body,
        out_type=out_type,
        mesh=_mesh(),
        compiler_params=_SC_PARAMS,
        scratch_types=(
            [pltpu.VMEM_SHARED((NP, D), jnp.float32)]
            + [pltpu.VMEM((2, CH), jnp.int32) for _ in range(3)]
            + [pltpu.VMEM((CH, D), jnp.float32) for _ in range(3)]
            + [pltpu.VMEM((RPT,), jnp.float32)]
            + [pltpu.SemaphoreType.DMA for _ in range(6)]
        ),
    )
    return f(gtab, gidx, sidx, zrows, dinv, s_in)


def _sc_gather_pairs(stab, pidx):
    def body(stab_h, pidx_h, out_h, idxp, prow, sem):
        c = lax.axis_index("c")
        s = lax.axis_index("s")
        base = (c * NS + s) * PPT
        for k in range(PPT // CH):
            off = base + k * CH
            pltpu.sync_copy(pidx_h.at[pl.ds(off, CH)], idxp)
            pltpu.async_copy(stab_h.at[idxp], prow, sem).wait()
            pltpu.sync_copy(prow, out_h.at[pl.ds(off, CH)])

    f = pl.kernel(
        body,
        out_type=jax.ShapeDtypeStruct((2 * BATCH, D), jnp.float32),
        mesh=_mesh(),
        compiler_params=_SC_PARAMS,
        scratch_types=[
            pltpu.VMEM((CH,), jnp.int32),
            pltpu.VMEM((CH, D), jnp.float32),
            pltpu.SemaphoreType.DMA,
        ],
    )
    return f(stab, pidx)


def _tc_decode(prows):
    def body(u_ref, i_ref, o_ref):
        o_ref[...] = jnp.sum(
            u_ref[...] * i_ref[...], axis=1, keepdims=True) * (1.0 / 16.0)

    return pl.pallas_call(
        body,
        grid=(2,),
        in_specs=[
            pl.BlockSpec((BATCH // 2, D), lambda i: (i, 0)),
            pl.BlockSpec((BATCH // 2, D), lambda i: (i + 2, 0)),
        ],
        out_specs=pl.BlockSpec((BATCH // 2, 1), lambda i: (i, 0)),
        out_shape=jax.ShapeDtypeStruct((BATCH, 1), jnp.float32),
    )(prows, prows)


def kernel(users, items, edge_list, emb_user, emb_item):
    users = users.astype(jnp.int32)
    items = items.astype(jnp.int32)
    el = edge_list.astype(jnp.int32)
    eu = el[:, 0]
    ei = el[:, 1]

    padw = NCH_IDX * CH - NE // NS

    spread = (jnp.arange(NS * padw, dtype=jnp.int32) % NU).reshape(NS, padw)
    sink = jnp.full((NS, padw), PAD_ROW, jnp.int32)

    def lanes(v, pad):
        return jnp.concatenate(
            [v.reshape(NS, NE // NS), pad], axis=1).reshape(NS * NCH_IDX, CH)

    gidx = jnp.concatenate([lanes(ei, spread), lanes(eu + NP, spread + NP)])
    sidx = jnp.concatenate([lanes(eu, sink), lanes(ei, sink)])

    zdeg = jnp.zeros((NP, 16), jnp.float32)
    zrows = jnp.zeros((NP, D), jnp.float32)
    ones_rows = jnp.ones((CH, 16), jnp.float32)

    x = jnp.concatenate([
        jnp.pad(emb_user, ((0, NP - NU), (0, 0))),
        jnp.pad(emb_item, ((0, NP - NI), (0, 0))),
    ])
    dinv, gtab = _sc_degree_prologue(sidx, zdeg, ones_rows, x)

    s = x
    for layer in range(NLAYERS):
        if layer < NLAYERS - 1:
            s, gtab = _sc_propagate(
                gtab, gidx, sidx, zrows, dinv, s, relu=True, need_g=True)
        else:
            (s,) = _sc_propagate(
                gtab, gidx, sidx, zrows, dinv, s, relu=False, need_g=False)

    pidx = jnp.concatenate([users, items + NP])
    prows = _sc_gather_pairs(s, pidx)
    scores = _tc_decode(prows)
    return scores.reshape(BATCH)


# speedup vs baseline: 37.4154x; 1.0004x over previous
import jax
import jax.numpy as jnp
from jax import lax
from jax.experimental import pallas as pl
from jax.experimental.pallas import tpu as pltpu
from jax.experimental.pallas import tpu_sc as plsc

NU = 25000
NI = 25000
D = 64
NE = 800000
NLAYERS = 3
BATCH = 4096

NC = 2
NS = 16
NP = 25088
RPT = NP // NS
CH = 128
NCH_PROP = 393
NCH_IDX = 396
PAD_ROW = NP - 1
PPT = 2 * BATCH // (NC * NS)

BLK = 3584


def _mesh():
    return plsc.VectorSubcoreMesh(
        core_axis_name="c", subcore_axis_name="s",
        num_cores=NC, num_subcores=NS)


_SC_PARAMS = pltpu.CompilerParams(
    use_tc_tiling_on_sc=False, needs_layout_passes=False)


def _rsqrt16(x):
    y = plsc.bitcast(jnp.int32(0x5F3759DF) - (plsc.bitcast(x, jnp.int32) >> 1),
                     jnp.float32)
    y = y * (1.5 - 0.5 * x * y * y)
    y = y * (1.5 - 0.5 * x * y * y)
    return y


def _sc_degree_prologue(sidx, zdeg, ones_rows, x):
    def body(sidx_h, zdeg_h, ones_h, x_h, dinv_h, g_h, deg_sp, obuf, xb, dvb,
             ib0, ib1, ib2, si0, si1, si2):
        ib = [ib0, ib1, ib2]
        si = [si0, si1, si2]
        c = lax.axis_index("c")
        s = lax.axis_index("s")
        sl = pl.ds(s * RPT, RPT)
        pltpu.sync_copy(zdeg_h.at[sl], deg_sp.at[sl])
        pltpu.sync_copy(ones_h, obuf)
        plsc.subcore_barrier()
        kb = (c * NS + s) * NCH_IDX

        def wait_idx(q):
            pltpu.make_async_copy(sidx_h.at[kb], ib[q], si[q]).wait()

        for q in range(3):
            pltpu.async_copy(sidx_h.at[kb + q], ib[q], si[q])

        def chunk(u, carry):
            j = kb + 3 * u
            for q in range(3):
                wait_idx(q)
                pltpu.sync_copy(obuf, deg_sp.at[ib[q]], add=True)
                pltpu.async_copy(sidx_h.at[j + q + 3], ib[q], si[q])
            return carry

        lax.fori_loop(0, NCH_PROP // 3, chunk, 0)
        for q in range(3):
            wait_idx(q)
        plsc.subcore_barrier()

        base = c * NP + s * RPT
        gbase = (1 - c) * NP + s * RPT
        iota16 = lax.iota(jnp.int32, 16)
        zero16 = jnp.zeros((16,), jnp.int32)

        def ep(k, carry):
            r0 = k * EPC
            db = obuf.at[pl.ds(0, EPC)]

            @pl.when(k > 0)
            def _():
                pltpu.make_async_copy(
                    dvb, dinv_h.at[pl.ds(base, EPC)], si0).wait()
                pltpu.make_async_copy(
                    xb, g_h.at[pl.ds(gbase, EPC)], si0).wait()

            pltpu.sync_copy(deg_sp.at[pl.ds(s * RPT + r0, EPC)], db)
            pltpu.sync_copy(x_h.at[pl.ds(base + r0, EPC)], xb)

            def grp(t, carry2):
                rr = t * 16
                deg = plsc.load_gather(obuf, [iota16 + rr, zero16])
                dvb[pl.ds(rr, 16)] = _rsqrt16(jnp.maximum(deg, 1.0))
                return carry2

            lax.fori_loop(0, EPC // 16, grp, 0)

            def row(i, carry2):
                dvs = plsc.load_gather(dvb, [jnp.full((16,), i, jnp.int32)])
                for q in range(D // 16):
                    cs = pl.ds(16 * q, 16)
                    xb[i, cs] = xb[i, cs] * dvs
                return carry2

            lax.fori_loop(0, EPC, row, 0)
            pltpu.async_copy(dvb, dinv_h.at[pl.ds(base + r0, EPC)], si0)
            pltpu.async_copy(xb, g_h.at[pl.ds(gbase + r0, EPC)], si0)
            return carry

        lax.fori_loop(0, RPT // EPC, ep, 0)
        pltpu.make_async_copy(dvb, dinv_h.at[pl.ds(base, EPC)], si0).wait()
        pltpu.make_async_copy(xb, g_h.at[pl.ds(gbase, EPC)], si0).wait()

    f = pl.kernel(
        body,
        out_type=[
            jax.ShapeDtypeStruct((NC * NP,), jnp.float32),
            jax.ShapeDtypeStruct((NC * NP, D), jnp.float32),
        ],
        mesh=_mesh(),
        compiler_params=_SC_PARAMS,
        scratch_types=(
            [pltpu.VMEM_SHARED((NP, 16), jnp.float32),
             pltpu.VMEM((CH, 16), jnp.float32),
             pltpu.VMEM((EPC, D), jnp.float32),
             pltpu.VMEM((EPC,), jnp.float32)]
            + [pltpu.VMEM((CH,), jnp.int32) for _ in range(3)]
            + [pltpu.SemaphoreType.DMA for _ in range(3)]
        ),
    )
    return f(sidx, zdeg, ones_rows, x)


EPC = 112


def _sc_propagate(gtab, gidx, sidx, zrows, dinv, s_in, relu, need_g):
    def body(gtab_h, gidx_h, sidx_h, zrows_h, dinv_h, sin_h, *rest):
        if need_g:
            sout_h, g_h = rest[0], rest[1]
            scratch = rest[2:]
        else:
            sout_h = rest[0]
            g_h = None
            scratch = rest[1:]
        (acc, ib0, ib1, ib2, rw0, rw1, rw2, dv,
         si0, si1, si2, sg0, sg1, sg2) = scratch
        ib = [ib0, ib1, ib2]
        rw = [rw0, rw1, rw2]
        si = [si0, si1, si2]
        sg = [sg0, sg1, sg2]
        c = lax.axis_index("c")
        s = lax.axis_index("s")
        sl = pl.ds(s * RPT, RPT)
        pltpu.sync_copy(zrows_h.at[sl], acc.at[sl])
        plsc.subcore_barrier()
        kb = (c * NS + s) * NCH_IDX

        def issue_idx(k, q):
            pltpu.async_copy(gidx_h.at[k], ib[q].at[0], si[q])
            pltpu.async_copy(sidx_h.at[k], ib[q].at[1], si[q])

        def wait_idx(q):
            pltpu.make_async_copy(gidx_h.at[pl.ds(kb, 2)], ib[q], si[q]).wait()

        def wait_gather(q):
            pltpu.make_async_copy(gtab_h.at[ib[q].at[0]], rw[q], sg[q]).wait()

        for q in range(3):
            issue_idx(kb + q, q)
        for q in range(2):
            wait_idx(q)
            pltpu.async_copy(gtab_h.at[ib[q].at[0]], rw[q], sg[q])

        def chunk(u, carry):
            j = kb + 3 * u
            for q in range(3):
                wait_gather(q)
                pltpu.sync_copy(rw[q], acc.at[ib[q].at[1]], add=True)
                issue_idx(j + q + 3, q)
                q2 = (q + 2) % 3
                wait_idx(q2)
                pltpu.async_copy(gtab_h.at[ib[q2].at[0]], rw[q2], sg[q2])
            return carry

        lax.fori_loop(0, NCH_PROP // 3, chunk, 0)
        wait_gather(0)
        wait_gather(1)
        wait_idx(2)
        plsc.subcore_barrier()

        base = c * NP + s * RPT
        gbase = (1 - c) * NP + s * RPT
        pltpu.sync_copy(dinv_h.at[pl.ds(base, RPT)], dv)

        nw = 2 if need_g else 1

        def ep(k, carry):
            r0 = k * EPC
            ea = rw0.at[pl.ds(0, EPC)]
            eb = rw1.at[pl.ds(0, EPC)]

            @pl.when(k > 0)
            def _():
                for _ in range(nw):
                    pltpu.make_async_copy(
                        eb, sout_h.at[pl.ds(base, EPC)], sg0).wait()

            pltpu.sync_copy(acc.at[pl.ds(s * RPT + r0, EPC)], ea)
            pltpu.sync_copy(sin_h.at[pl.ds(base + r0, EPC)], eb)

            def row(i, carry2):
                dvs = plsc.load_gather(
                    dv, [jnp.full((16,), r0 + i, jnp.int32)])
                for q in range(D // 16):
                    cs = pl.ds(16 * q, 16)
                    h = rw0[i, cs] * dvs
                    if relu:
                        h = jnp.maximum(h, 0.0)
                    rw1[i, cs] = rw1[i, cs] + h
                    if need_g:
                        rw0[i, cs] = h * dvs
                return carry2

            lax.fori_loop(0, EPC, row, 0)
            pltpu.async_copy(eb, sout_h.at[pl.ds(base + r0, EPC)], sg0)
            if need_g:
                pltpu.async_copy(ea, g_h.at[pl.ds(gbase + r0, EPC)], sg0)
            return carry

        lax.fori_loop(0, RPT // EPC, ep, 0)
        for _ in range(nw):
            pltpu.make_async_copy(
                rw1.at[pl.ds(0, EPC)], sout_h.at[pl.ds(base, EPC)], sg0).wait()

    out_type = [jax.ShapeDtypeStruct((NC * NP, D), jnp.float32)]
    if need_g:
        out_type.append(jax.ShapeDtypeStruct((NC * NP, D), jnp.float32))
    f = pl.kernel(
        body,
        out_type=out_type,
        mesh=_mesh(),
        compiler_params=_SC_PARAMS,
        scratch_types=(
            [pltpu.VMEM_SHARED((NP, D), jnp.float32)]
            + [pltpu.VMEM((2, CH), jnp.int32) for _ in range(3)]
            + [pltpu.VMEM((CH, D), jnp.float32) for _ in range(3)]
            + [pltpu.VMEM((RPT,), jnp.float32)]
            + [pltpu.SemaphoreType.DMA for _ in range(6)]
        ),
    )
    return f(gtab, gidx, sidx, zrows, dinv, s_in)


def _sc_gather_pairs(stab, pidx):
    def body(stab_h, pidx_h, out_h, idxp, prow, sem):
        c = lax.axis_index("c")
        s = lax.axis_index("s")
        base = (c * NS + s) * PPT
        for k in range(PPT // CH):
            off = base + k * CH
            pltpu.sync_copy(pidx_h.at[pl.ds(off, CH)], idxp)
            pltpu.async_copy(stab_h.at[idxp], prow, sem).wait()
            pltpu.sync_copy(prow, out_h.at[pl.ds(off, CH)])

    f = pl.kernel(
        body,
        out_type=jax.ShapeDtypeStruct((2 * BATCH, D), jnp.float32),
        mesh=_mesh(),
        compiler_params=_SC_PARAMS,
        scratch_types=[
            pltpu.VMEM((CH,), jnp.int32),
            pltpu.VMEM((CH, D), jnp.float32),
            pltpu.SemaphoreType.DMA,
        ],
    )
    return f(stab, pidx)


def _tc_decode(prows):
    def body(u_ref, i_ref, o_ref):
        o_ref[...] = jnp.sum(
            u_ref[...] * i_ref[...], axis=1, keepdims=True) * (1.0 / 16.0)

    return pl.pallas_call(
        body,
        grid=(2,),
        in_specs=[
            pl.BlockSpec((BATCH // 2, D), lambda i: (i, 0)),
            pl.BlockSpec((BATCH // 2, D), lambda i: (i + 2, 0)),
        ],
        out_specs=pl.BlockSpec((BATCH // 2, 1), lambda i: (i, 0)),
        out_shape=jax.ShapeDtypeStruct((BATCH, 1), jnp.float32),
    )(prows, prows)


def kernel(users, items, edge_list, emb_user, emb_item):
    users = users.astype(jnp.int32)
    items = items.astype(jnp.int32)
    el = edge_list.astype(jnp.int32)
    eu = el[:, 0]
    ei = el[:, 1]

    padw = NCH_IDX * CH - NE // NS

    spread = (jnp.arange(NS * padw, dtype=jnp.int32) % NU).reshape(NS, padw)
    sink = jnp.full((NS, padw), PAD_ROW, jnp.int32)

    def lanes(v, pad):
        return jnp.concatenate(
            [v.reshape(NS, NE // NS), pad], axis=1).reshape(NS * NCH_IDX, CH)

    gidx = jnp.concatenate([lanes(ei, spread), lanes(eu + NP, spread + NP)])
    sidx = jnp.concatenate([lanes(eu, sink), lanes(ei, sink)])

    zdeg = jnp.zeros((NP, 16), jnp.float32)
    zrows = jnp.zeros((NP, D), jnp.float32)
    ones_rows = jnp.ones((CH, 16), jnp.float32)

    x = jnp.concatenate([
        jnp.pad(emb_user, ((0, NP - NU), (0, 0))),
        jnp.pad(emb_item, ((0, NP - NI), (0, 0))),
    ])
    dinv, gtab = _sc_degree_prologue(sidx, zdeg, ones_rows, x)

    s = x
    for layer in range(NLAYERS):
        if layer < NLAYERS - 1:
            s, gtab = _sc_propagate(
                gtab, gidx, sidx, zrows, dinv, s, relu=True, need_g=True)
        else:
            (s,) = _sc_propagate(
                gtab, gidx, sidx, zrows, dinv, s, relu=False, need_g=False)

    pidx = jnp.concatenate([users, items + NP])
    prows = _sc_gather_pairs(s, pidx)
    scores = _tc_decode(prows)
    return scores.reshape(BATCH)
